# baseline (speedup 1.0000x reference)
import functools

import jax
import jax.numpy as jnp
from jax import lax
from jax.experimental import pallas as pl
from jax.experimental.pallas import tpu as pltpu
from jax.experimental.pallas import tpu_sc as plsc

_NC = 2
_NS = 16
_NW = _NC * _NS


def _sc_gather(V, lin_table, idx, chunk=256, nbuf=3):
    BM = idx.shape[0]
    K = V.shape[1]
    lin_flat = lin_table.reshape(-1)
    per_w = BM // _NW
    n_chunks = per_w // chunk
    prefire = nbuf - 1
    assert BM % _NW == 0 and per_w % chunk == 0 and chunk % 8 == 0
    idx2d = idx.reshape(_NW * n_chunks, chunk)

    mesh = plsc.VectorSubcoreMesh(
        core_axis_name="c", subcore_axis_name="s",
        num_cores=_NC, num_subcores=_NS,
    )

    @functools.partial(
        pl.kernel,
        mesh=mesh,
        compiler_params=pltpu.CompilerParams(use_tc_tiling_on_sc=False),
        out_type=(
            jax.ShapeDtypeStruct((BM, K), jnp.float32),
            jax.ShapeDtypeStruct((_NW, n_chunks, chunk), jnp.float32),
        ),
        scratch_types=[
            pltpu.VMEM((n_chunks, chunk), jnp.int32),
            pltpu.VMEM((n_chunks, chunk), jnp.float32),
            pltpu.VMEM((nbuf, chunk, K), jnp.float32),
            pltpu.SemaphoreType.DMA,
            pltpu.SemaphoreType.DMA,
            pltpu.SemaphoreType.DMA,
        ],
    )
    def gather_kernel(v_hbm, lin_hbm, idx_hbm, emb_out, lin_out,
                      idx_v, lin_v, rows_v, sem_g, sem_o, sem_l):
        wid = lax.axis_index("s") * _NC + lax.axis_index("c")
        base = wid * per_w
        pltpu.sync_copy(idx_hbm.at[pl.ds(wid * n_chunks, n_chunks)], idx_v)
        for g in range(n_chunks):
            pltpu.async_copy(lin_hbm.at[idx_v.at[g]], lin_v.at[g], sem_l)

        def fire(g):
            pltpu.async_copy(v_hbm.at[idx_v.at[g]], rows_v.at[g % nbuf],
                             sem_g)

        def wait_gather(g):
            pltpu.make_async_copy(v_hbm.at[idx_v.at[g]],
                                  rows_v.at[g % nbuf], sem_g).wait()

        def copy_out(g):
            pltpu.async_copy(rows_v.at[g % nbuf],
                             emb_out.at[pl.ds(base + g * chunk, chunk)],
                             sem_o)

        def wait_out(g):
            pltpu.make_async_copy(rows_v.at[g % nbuf],
                                  emb_out.at[pl.ds(base + g * chunk, chunk)],
                                  sem_o).wait()

        for g in range(prefire):
            fire(g)
        for g in range(n_chunks):
            wait_gather(g)
            copy_out(g)
            f = g + prefire
            if f < n_chunks:
                if f >= nbuf:
                    wait_out(f - nbuf)
                fire(f)
        for g in range(n_chunks - nbuf, n_chunks):
            wait_out(g)
        for g in range(n_chunks):
            pltpu.make_async_copy(lin_hbm.at[idx_v.at[g]], lin_v.at[g],
                                  sem_l).wait()
        pltpu.sync_copy(lin_v, lin_out.at[wid])

    return gather_kernel(V, lin_flat, idx2d)


def _mlp_block(embd_ref, dense_ref, linv_ref, w0e_ref, w0d_ref, b0_ref,
               w1_ref, b1_ref, w2_ref, b2_ref, wh_ref, wli_ref, out_ref,
               *, m_fields, k_dim):
    e = embd_ref[...]
    s = e[:, 0:k_dim]
    for m in range(1, m_fields):
        s = s + e[:, m * k_dim:(m + 1) * k_dim]
    sum_sq = jnp.sum(e * e, axis=1, keepdims=True)
    sq_sum = jnp.sum(s * s, axis=1, keepdims=True)
    inter = 0.5 * (sq_sum - sum_sq)
    lin = jnp.sum(linv_ref[...], axis=1, keepdims=True)

    h = e @ w0e_ref[...] + dense_ref[...] @ w0d_ref[...] + b0_ref[...]
    h = jnp.maximum(h, 0.0)
    h = jnp.maximum(h @ w1_ref[...] + b1_ref[...], 0.0)
    h = jnp.maximum(h @ w2_ref[...] + b2_ref[...], 0.0)
    wli = wli_ref[...]
    z = (h @ wh_ref[...] + lin * wli[0, 0] + inter * wli[0, 1]
         + wli[0, 2])
    out_ref[...] = jax.nn.sigmoid(z)


def _tc_mlp(embd_flat, dense, linv, W0, b0, W1, b1, W2, b2, Wfc, bfc,
            block_b, interpret=False):
    B, MK = embd_flat.shape
    M = linv.shape[1]
    K = MK // M
    D = dense.shape[1]
    H0, H1, H2 = W0.shape[1], W1.shape[1], W2.shape[1]
    w0e = W0[:MK]
    w0d = W0[MK:]
    wh = Wfc[2:]
    wli = jnp.concatenate([Wfc[0:1, 0], Wfc[1:2, 0], bfc]).reshape(1, 3)
    grid = (B // block_b,)

    out = pl.pallas_call(
        functools.partial(_mlp_block, m_fields=M, k_dim=K),
        grid=grid,
        in_specs=[
            pl.BlockSpec((block_b, MK), lambda i: (i, 0)),
            pl.BlockSpec((block_b, D), lambda i: (i, 0)),
            pl.BlockSpec((block_b, M), lambda i: (i, 0)),
            pl.BlockSpec((MK, H0), lambda i: (0, 0)),
            pl.BlockSpec((D, H0), lambda i: (0, 0)),
            pl.BlockSpec((1, H0), lambda i: (0, 0)),
            pl.BlockSpec((H0, H1), lambda i: (0, 0)),
            pl.BlockSpec((1, H1), lambda i: (0, 0)),
            pl.BlockSpec((H1, H2), lambda i: (0, 0)),
            pl.BlockSpec((1, H2), lambda i: (0, 0)),
            pl.BlockSpec((H2, 1), lambda i: (0, 0)),
            pl.BlockSpec((1, 3), lambda i: (0, 0)),
        ],
        out_specs=pl.BlockSpec((block_b, 1), lambda i: (i, 0)),
        out_shape=jax.ShapeDtypeStruct((B, 1), jnp.float32),
        interpret=interpret,
    )(embd_flat, dense, linv, w0e, w0d, b0.reshape(1, H0), W1,
      b1.reshape(1, H1), W2, b2.reshape(1, H2), wh, wli)
    return out[:, 0]


def kernel(cat_features, dense_features, lin_table, V, W0, b0, W1, b1,
           W2, b2, Wfc, bfc):
    B, M = cat_features.shape
    K = V.shape[1]
    idx = cat_features.reshape(-1).astype(jnp.int32)
    P = 2
    Bs = B // P
    outs = []
    for p in range(P):
        idx_p = lax.dynamic_slice_in_dim(idx, p * Bs * M, Bs * M)
        emb_rows, lin_rows = _sc_gather(V, lin_table, idx_p,
                                        chunk=(Bs * M // _NW) // 13)
        embd_flat = emb_rows.reshape(Bs, M * K)
        linv = lin_rows.reshape(Bs, M)
        dense_p = lax.dynamic_slice_in_dim(dense_features, p * Bs, Bs)
        outs.append(_tc_mlp(embd_flat, dense_p, linv, W0, b0, W1, b1, W2,
                            b2, Wfc, bfc, block_b=512))
    return jnp.concatenate(outs, axis=0)


# speedup vs baseline: 1.0634x; 1.0634x over previous
import functools

import jax
import jax.numpy as jnp
from jax import lax
from jax.experimental import pallas as pl
from jax.experimental.pallas import tpu as pltpu
from jax.experimental.pallas import tpu_sc as plsc

_NC = 2
_NS = 16
_NW = _NC * _NS


def _sc_gather(V, lin_table, idx, chunk=256, nbuf=3):
    BM = idx.shape[0]
    K = V.shape[1]
    lin_flat = lin_table.reshape(-1)
    per_w = BM // _NW
    n_chunks = per_w // chunk
    prefire = nbuf - 1
    assert BM % _NW == 0 and per_w % chunk == 0 and chunk % 8 == 0
    idx2d = idx.reshape(_NW * n_chunks, chunk)

    mesh = plsc.VectorSubcoreMesh(
        core_axis_name="c", subcore_axis_name="s",
        num_cores=_NC, num_subcores=_NS,
    )

    @functools.partial(
        pl.kernel,
        mesh=mesh,
        compiler_params=pltpu.CompilerParams(use_tc_tiling_on_sc=False),
        out_type=(
            jax.ShapeDtypeStruct((BM, K), jnp.float32),
            jax.ShapeDtypeStruct((_NW, n_chunks, chunk), jnp.float32),
        ),
        scratch_types=[
            pltpu.VMEM((n_chunks, chunk), jnp.int32),
            pltpu.VMEM((n_chunks, chunk), jnp.float32),
            pltpu.VMEM((nbuf, chunk, K), jnp.float32),
            pltpu.SemaphoreType.DMA,
            pltpu.SemaphoreType.DMA,
            pltpu.SemaphoreType.DMA,
        ],
    )
    def gather_kernel(v_hbm, lin_hbm, idx_hbm, emb_out, lin_out,
                      idx_v, lin_v, rows_v, sem_g, sem_o, sem_l):
        wid = lax.axis_index("s") * _NC + lax.axis_index("c")
        base = wid * per_w
        pltpu.sync_copy(idx_hbm.at[pl.ds(wid * n_chunks, n_chunks)], idx_v)
        for g in range(n_chunks):
            pltpu.async_copy(lin_hbm.at[idx_v.at[g]], lin_v.at[g], sem_l)

        def fire(g):
            pltpu.async_copy(v_hbm.at[idx_v.at[g]], rows_v.at[g % nbuf],
                             sem_g)

        def wait_gather(g):
            pltpu.make_async_copy(v_hbm.at[idx_v.at[g]],
                                  rows_v.at[g % nbuf], sem_g).wait()

        def copy_out(g):
            pltpu.async_copy(rows_v.at[g % nbuf],
                             emb_out.at[pl.ds(base + g * chunk, chunk)],
                             sem_o)

        def wait_out(g):
            pltpu.make_async_copy(rows_v.at[g % nbuf],
                                  emb_out.at[pl.ds(base + g * chunk, chunk)],
                                  sem_o).wait()

        for g in range(prefire):
            fire(g)
        for g in range(n_chunks):
            wait_gather(g)
            copy_out(g)
            f = g + prefire
            if f < n_chunks:
                if f >= nbuf:
                    wait_out(f - nbuf)
                fire(f)
        for g in range(n_chunks - nbuf, n_chunks):
            wait_out(g)
        for g in range(n_chunks):
            pltpu.make_async_copy(lin_hbm.at[idx_v.at[g]], lin_v.at[g],
                                  sem_l).wait()
        pltpu.sync_copy(lin_v, lin_out.at[wid])

    return gather_kernel(V, lin_flat, idx2d)


def _mlp_block(embd_ref, dense_ref, linv_ref, w0e_ref, w0d_ref, b0_ref,
               w1_ref, b1_ref, w2_ref, b2_ref, wh_ref, wli_ref, out_ref,
               *, m_fields, k_dim):
    e = embd_ref[...]
    s = e[:, 0:k_dim]
    for m in range(1, m_fields):
        s = s + e[:, m * k_dim:(m + 1) * k_dim]
    sum_sq = jnp.sum(e * e, axis=1, keepdims=True)
    sq_sum = jnp.sum(s * s, axis=1, keepdims=True)
    inter = 0.5 * (sq_sum - sum_sq)
    lin = jnp.sum(linv_ref[...], axis=1, keepdims=True)

    f32 = jnp.float32
    bf = jnp.bfloat16
    dot = functools.partial(jnp.dot, preferred_element_type=f32)
    h = (dot(e.astype(bf), w0e_ref[...]) +
         dot(dense_ref[...].astype(bf), w0d_ref[...]) + b0_ref[...])
    h = jnp.maximum(h, 0.0)
    h = jnp.maximum(dot(h.astype(bf), w1_ref[...]) + b1_ref[...], 0.0)
    h = jnp.maximum(dot(h.astype(bf), w2_ref[...]) + b2_ref[...], 0.0)
    wli = wli_ref[...]
    z = (h @ wh_ref[...] + lin * wli[0, 0] + inter * wli[0, 1]
         + wli[0, 2])
    out_ref[...] = jax.nn.sigmoid(z)


def _tc_mlp(embd_flat, dense, linv, W0, b0, W1, b1, W2, b2, Wfc, bfc,
            block_b, interpret=False):
    B, MK = embd_flat.shape
    M = linv.shape[1]
    K = MK // M
    D = dense.shape[1]
    H0, H1, H2 = W0.shape[1], W1.shape[1], W2.shape[1]
    bf = jnp.bfloat16
    w0e = W0[:MK].astype(bf)
    w0d = W0[MK:].astype(bf)
    W1 = W1.astype(bf)
    W2 = W2.astype(bf)
    wh = Wfc[2:]
    wli = jnp.concatenate([Wfc[0:1, 0], Wfc[1:2, 0], bfc]).reshape(1, 3)
    grid = (B // block_b,)

    out = pl.pallas_call(
        functools.partial(_mlp_block, m_fields=M, k_dim=K),
        grid=grid,
        in_specs=[
            pl.BlockSpec((block_b, MK), lambda i: (i, 0)),
            pl.BlockSpec((block_b, D), lambda i: (i, 0)),
            pl.BlockSpec((block_b, M), lambda i: (i, 0)),
            pl.BlockSpec((MK, H0), lambda i: (0, 0)),
            pl.BlockSpec((D, H0), lambda i: (0, 0)),
            pl.BlockSpec((1, H0), lambda i: (0, 0)),
            pl.BlockSpec((H0, H1), lambda i: (0, 0)),
            pl.BlockSpec((1, H1), lambda i: (0, 0)),
            pl.BlockSpec((H1, H2), lambda i: (0, 0)),
            pl.BlockSpec((1, H2), lambda i: (0, 0)),
            pl.BlockSpec((H2, 1), lambda i: (0, 0)),
            pl.BlockSpec((1, 3), lambda i: (0, 0)),
        ],
        out_specs=pl.BlockSpec((block_b, 1), lambda i: (i, 0)),
        out_shape=jax.ShapeDtypeStruct((B, 1), jnp.float32),
        interpret=interpret,
    )(embd_flat, dense, linv, w0e, w0d, b0.reshape(1, H0), W1,
      b1.reshape(1, H1), W2, b2.reshape(1, H2), wh, wli)
    return out[:, 0]


def kernel(cat_features, dense_features, lin_table, V, W0, b0, W1, b1,
           W2, b2, Wfc, bfc):
    B, M = cat_features.shape
    K = V.shape[1]
    idx = cat_features.reshape(-1).astype(jnp.int32)
    emb_rows, lin_rows = _sc_gather(V, lin_table, idx)
    embd_flat = emb_rows.reshape(B, M * K)
    linv = lin_rows.reshape(B, M)
    return _tc_mlp(embd_flat, dense_features, linv, W0, b0, W1, b1, W2,
                   b2, Wfc, bfc, block_b=512)


# speedup vs baseline: 1.6158x; 1.5195x over previous
import functools

import jax
import jax.numpy as jnp
from jax import lax
from jax.experimental import pallas as pl
from jax.experimental.pallas import tpu as pltpu
from jax.experimental.pallas import tpu_sc as plsc

_NC = 2
_NS = 16
_NW = _NC * _NS


def _sc_gather(V, lin_table, idx_wm, M, B, nbuf=4):
    K = V.shape[1]
    chunk = B // _NW
    lin_flat = lin_table.reshape(-1)
    assert chunk % 8 == 0

    mesh = plsc.VectorSubcoreMesh(
        core_axis_name="c", subcore_axis_name="s",
        num_cores=_NC, num_subcores=_NS,
    )

    @functools.partial(
        pl.kernel,
        mesh=mesh,
        compiler_params=pltpu.CompilerParams(use_tc_tiling_on_sc=False),
        out_type=(
            jax.ShapeDtypeStruct((M, B, K), jnp.float32),
            jax.ShapeDtypeStruct((M, _NW, chunk), jnp.float32),
        ),
        scratch_types=[
            pltpu.VMEM((M, chunk), jnp.int32),
            pltpu.VMEM((M, chunk), jnp.float32),
            pltpu.VMEM((nbuf, chunk, K), jnp.float32),
            pltpu.SemaphoreType.DMA,
            pltpu.SemaphoreType.DMA,
            pltpu.SemaphoreType.DMA,
        ],
    )
    def gather_kernel(v_hbm, lin_hbm, idx_hbm, emb_out, lin_out,
                      idx_v, lin_v, rows_v, sem_g, sem_o, sem_l):
        wid = lax.axis_index("s") * _NC + lax.axis_index("c")
        pltpu.sync_copy(idx_hbm.at[wid], idx_v)
        for m in range(M):
            pltpu.async_copy(lin_hbm.at[idx_v.at[m]], lin_v.at[m], sem_l)

        def fire(m):
            pltpu.async_copy(v_hbm.at[idx_v.at[m]], rows_v.at[m % nbuf],
                             sem_g)

        def wait_gather(m):
            pltpu.make_async_copy(v_hbm.at[idx_v.at[m]],
                                  rows_v.at[m % nbuf], sem_g).wait()

        def copy_out(m):
            pltpu.async_copy(rows_v.at[m % nbuf],
                             emb_out.at[m, pl.ds(wid * chunk, chunk)],
                             sem_o)

        def wait_out(m):
            pltpu.make_async_copy(rows_v.at[m % nbuf],
                                  emb_out.at[m, pl.ds(wid * chunk, chunk)],
                                  sem_o).wait()

        prefire = nbuf - 1
        for m in range(prefire):
            fire(m)
        for m in range(M):
            wait_gather(m)
            copy_out(m)
            f = m + prefire
            if f < M:
                if f >= nbuf:
                    wait_out(f - nbuf)
                fire(f)
        for m in range(max(M - nbuf, 0), M):
            wait_out(m)
        for m in range(M):
            pltpu.make_async_copy(lin_hbm.at[idx_v.at[m]], lin_v.at[m],
                                  sem_l).wait()
        pltpu.sync_copy(lin_v, lin_out.at[pl.ds(0, M), wid])

    return gather_kernel(V, lin_flat, idx_wm)


def _mlp_block(e3_ref, dense_ref, linv_ref, w0_ref, b0_ref,
               w1_ref, b1_ref, w2_ref, b2_ref, wh_ref, wli_ref, out_ref,
               *, m_fields, k_dim):
    planes = [e3_ref[m] for m in range(m_fields)]
    s = planes[0]
    ss = jnp.sum(planes[0] * planes[0], axis=1, keepdims=True)
    for m in range(1, m_fields):
        p = planes[m]
        s = s + p
        ss = ss + jnp.sum(p * p, axis=1, keepdims=True)
    inter = 0.5 * (jnp.sum(s * s, axis=1, keepdims=True) - ss)
    lin = jnp.sum(linv_ref[...], axis=1, keepdims=True)

    e = jnp.concatenate(planes, axis=1)
    mk = m_fields * k_dim
    h = (e @ w0_ref[0:mk, :] + dense_ref[...] @ w0_ref[mk:, :]
         + b0_ref[...])
    h = jnp.maximum(h, 0.0)
    h = jnp.maximum(h @ w1_ref[...] + b1_ref[...], 0.0)
    h = jnp.maximum(h @ w2_ref[...] + b2_ref[...], 0.0)
    wli = wli_ref[...]
    z = (h @ wh_ref[...] + lin * wli[0, 0] + inter * wli[0, 1]
         + wli[0, 2])
    out_ref[...] = jax.nn.sigmoid(z)


def _tc_mlp(e3, dense, linv, W0, b0, W1, b1, W2, b2, Wfc, bfc,
            block_b, interpret=False):
    M, B, K = e3.shape
    D = dense.shape[1]
    H0, H1, H2 = W0.shape[1], W1.shape[1], W2.shape[1]
    wh = Wfc[2:]
    wli = jnp.concatenate([Wfc[0:1, 0], Wfc[1:2, 0], bfc]).reshape(1, 3)
    grid = (B // block_b,)

    out = pl.pallas_call(
        functools.partial(_mlp_block, m_fields=M, k_dim=K),
        grid=grid,
        in_specs=[
            pl.BlockSpec((M, block_b, K), lambda i: (0, i, 0)),
            pl.BlockSpec((block_b, D), lambda i: (i, 0)),
            pl.BlockSpec((block_b, M), lambda i: (i, 0)),
            pl.BlockSpec((M * K + D, H0), lambda i: (0, 0)),
            pl.BlockSpec((1, H0), lambda i: (0, 0)),
            pl.BlockSpec((H0, H1), lambda i: (0, 0)),
            pl.BlockSpec((1, H1), lambda i: (0, 0)),
            pl.BlockSpec((H1, H2), lambda i: (0, 0)),
            pl.BlockSpec((1, H2), lambda i: (0, 0)),
            pl.BlockSpec((H2, 1), lambda i: (0, 0)),
            pl.BlockSpec((1, 3), lambda i: (0, 0)),
        ],
        out_specs=pl.BlockSpec((block_b, 1), lambda i: (i, 0)),
        out_shape=jax.ShapeDtypeStruct((B, 1), jnp.float32),
        interpret=interpret,
    )(e3, dense, linv, W0, b0.reshape(1, H0), W1,
      b1.reshape(1, H1), W2, b2.reshape(1, H2), wh, wli)
    return out[:, 0]


def kernel(cat_features, dense_features, lin_table, V, W0, b0, W1, b1,
           W2, b2, Wfc, bfc):
    B, M = cat_features.shape
    K = V.shape[1]
    chunk = B // _NW
    idx_wm = (cat_features.astype(jnp.int32)
              .reshape(_NW, chunk, M).transpose(0, 2, 1))
    e3, lin_mw = _sc_gather(V, lin_table, idx_wm, M, B)
    linv = lin_mw.reshape(M, B).T
    return _tc_mlp(e3, dense_features, linv, W0, b0, W1, b1, W2,
                   b2, Wfc, bfc, block_b=512)


# speedup vs baseline: 1.6400x; 1.0150x over previous
import functools

import jax
import jax.numpy as jnp
from jax import lax
from jax.experimental import pallas as pl
from jax.experimental.pallas import tpu as pltpu
from jax.experimental.pallas import tpu_sc as plsc

_NC = 2
_NS = 16
_NW = _NC * _NS


def _sc_gather(V, lin_table, idx_wm, M, B, nbuf=4):
    K = V.shape[1]
    chunk = B // _NW
    lin_flat = lin_table.reshape(-1)
    assert chunk % 8 == 0

    mesh = plsc.VectorSubcoreMesh(
        core_axis_name="c", subcore_axis_name="s",
        num_cores=_NC, num_subcores=_NS,
    )

    @functools.partial(
        pl.kernel,
        mesh=mesh,
        compiler_params=pltpu.CompilerParams(use_tc_tiling_on_sc=False),
        out_type=(
            jax.ShapeDtypeStruct((M, B, K), jnp.float32),
            jax.ShapeDtypeStruct((M, _NW, chunk), jnp.float32),
        ),
        scratch_types=[
            pltpu.VMEM((M, chunk), jnp.int32),
            pltpu.VMEM((M, chunk), jnp.float32),
            pltpu.VMEM((nbuf, chunk, K), jnp.float32),
            pltpu.SemaphoreType.DMA,
            pltpu.SemaphoreType.DMA,
            pltpu.SemaphoreType.DMA,
        ],
    )
    def gather_kernel(v_hbm, lin_hbm, idx_hbm, emb_out, lin_out,
                      idx_v, lin_v, rows_v, sem_g, sem_o, sem_l):
        wid = lax.axis_index("s") * _NC + lax.axis_index("c")
        pltpu.sync_copy(idx_hbm.at[wid], idx_v)
        for m in range(M):
            pltpu.async_copy(lin_hbm.at[idx_v.at[m]], lin_v.at[m], sem_l)

        def fire(m):
            pltpu.async_copy(v_hbm.at[idx_v.at[m]], rows_v.at[m % nbuf],
                             sem_g)

        def wait_gather(m):
            pltpu.make_async_copy(v_hbm.at[idx_v.at[m]],
                                  rows_v.at[m % nbuf], sem_g).wait()

        def copy_out(m):
            pltpu.async_copy(rows_v.at[m % nbuf],
                             emb_out.at[m, pl.ds(wid * chunk, chunk)],
                             sem_o)

        def wait_out(m):
            pltpu.make_async_copy(rows_v.at[m % nbuf],
                                  emb_out.at[m, pl.ds(wid * chunk, chunk)],
                                  sem_o).wait()

        prefire = nbuf - 1
        for m in range(prefire):
            fire(m)
        for m in range(M):
            wait_gather(m)
            copy_out(m)
            f = m + prefire
            if f < M:
                if f >= nbuf:
                    wait_out(f - nbuf)
                fire(f)
        for m in range(max(M - nbuf, 0), M):
            wait_out(m)
        for m in range(M):
            pltpu.make_async_copy(lin_hbm.at[idx_v.at[m]], lin_v.at[m],
                                  sem_l).wait()
        pltpu.sync_copy(lin_v, lin_out.at[pl.ds(0, M), wid])

    return gather_kernel(V, lin_flat, idx_wm)


def _mlp_block(e3_ref, dense_ref, linv_ref, w0_ref, b0_ref,
               w1_ref, b1_ref, w2_ref, b2_ref, wh_ref, wli_ref, out_ref,
               *, m_fields, k_dim):
    planes = [e3_ref[m] for m in range(m_fields)]
    s = planes[0]
    ss = jnp.sum(planes[0] * planes[0], axis=1, keepdims=True)
    for m in range(1, m_fields):
        p = planes[m]
        s = s + p
        ss = ss + jnp.sum(p * p, axis=1, keepdims=True)
    inter = 0.5 * (jnp.sum(s * s, axis=1, keepdims=True) - ss)
    lin = jnp.sum(linv_ref[...], axis=1, keepdims=True)

    e = jnp.concatenate(planes, axis=1)
    mk = m_fields * k_dim
    h = (e @ w0_ref[0:mk, :] + dense_ref[...] @ w0_ref[mk:, :]
         + b0_ref[...])
    h = jnp.maximum(h, 0.0)
    h = jnp.maximum(h @ w1_ref[...] + b1_ref[...], 0.0)
    h = jnp.maximum(h @ w2_ref[...] + b2_ref[...], 0.0)
    wli = wli_ref[...]
    z = (h @ wh_ref[...] + lin * wli[0, 0] + inter * wli[0, 1]
         + wli[0, 2])
    out_ref[...] = jax.nn.sigmoid(z)


def _tc_mlp(e3, dense, linv, W0, b0, W1, b1, W2, b2, Wfc, bfc,
            block_b, interpret=False):
    M, B, K = e3.shape
    D = dense.shape[1]
    H0, H1, H2 = W0.shape[1], W1.shape[1], W2.shape[1]
    wh = Wfc[2:]
    wli = jnp.concatenate([Wfc[0:1, 0], Wfc[1:2, 0], bfc]).reshape(1, 3)
    grid = (B // block_b,)

    out = pl.pallas_call(
        functools.partial(_mlp_block, m_fields=M, k_dim=K),
        grid=grid,
        in_specs=[
            pl.BlockSpec((M, block_b, K), lambda i: (0, i, 0)),
            pl.BlockSpec((block_b, D), lambda i: (i, 0)),
            pl.BlockSpec((block_b, M), lambda i: (i, 0)),
            pl.BlockSpec((M * K + D, H0), lambda i: (0, 0)),
            pl.BlockSpec((1, H0), lambda i: (0, 0)),
            pl.BlockSpec((H0, H1), lambda i: (0, 0)),
            pl.BlockSpec((1, H1), lambda i: (0, 0)),
            pl.BlockSpec((H1, H2), lambda i: (0, 0)),
            pl.BlockSpec((1, H2), lambda i: (0, 0)),
            pl.BlockSpec((H2, 1), lambda i: (0, 0)),
            pl.BlockSpec((1, 3), lambda i: (0, 0)),
        ],
        out_specs=pl.BlockSpec((block_b, 1), lambda i: (i, 0)),
        out_shape=jax.ShapeDtypeStruct((B, 1), jnp.float32),
        interpret=interpret,
    )(e3, dense, linv, W0, b0.reshape(1, H0), W1,
      b1.reshape(1, H1), W2, b2.reshape(1, H2), wh, wli)
    return out[:, 0]


def kernel(cat_features, dense_features, lin_table, V, W0, b0, W1, b1,
           W2, b2, Wfc, bfc):
    B, M = cat_features.shape
    K = V.shape[1]
    P = 2
    Bs = B // P
    chunk = Bs // _NW
    cat32 = cat_features.astype(jnp.int32)
    outs = []
    for p in range(P):
        cat_p = lax.dynamic_slice_in_dim(cat32, p * Bs, Bs)
        idx_wm = cat_p.reshape(_NW, chunk, M).transpose(0, 2, 1)
        e3, lin_mw = _sc_gather(V, lin_table, idx_wm, M, Bs)
        linv = lin_mw.reshape(M, Bs).T
        dense_p = lax.dynamic_slice_in_dim(dense_features, p * Bs, Bs)
        outs.append(_tc_mlp(e3, dense_p, linv, W0, b0, W1, b1, W2,
                            b2, Wfc, bfc, block_b=512))
    return jnp.concatenate(outs, axis=0)
